# baseline (speedup 1.0000x reference)
import functools

import jax
import jax.numpy as jnp
from jax import lax
from jax.experimental import pallas as pl
from jax.experimental.pallas import tpu as pltpu
from jax.experimental.pallas import tpu_sc as plsc

VOCAB = 1000000
DIM = 64
BATCH = 16384
CTX = 20
NEG = 20

NC = 2
NS = 16
NW = NC * NS
BPW = BATCH // NW
CB = 32
NCHUNK = BPW // CB
ROWS = CB * CTX
TPG = ROWS // 128
ND = DIM // 16

_SC_MESH = plsc.VectorSubcoreMesh(core_axis_name="c", subcore_axis_name="s",
                                  num_cores=NC, num_subcores=NS)
_SC_PARAMS = pltpu.CompilerParams(needs_layout_passes=False,
                                  use_tc_tiling_on_sc=False)


def _mo8(x):
    return pl.multiple_of(x, 8)


def _k1_body(ctx_idx_hbm, in_emb, ctxv_out,
             ctx_idx_v, ctx_rows_v, cv_v, sem):
    wid = lax.axis_index("s") * NC + lax.axis_index("c")
    pltpu.sync_copy(ctx_idx_hbm.at[pl.ds(_mo8(wid * BPW * CTX), BPW * CTX)], ctx_idx_v)

    def chunk_body(c, carry):
        off = _mo8(c * ROWS)
        cps = [pltpu.async_copy(
            in_emb.at[ctx_idx_v.at[pl.ds(off + t * 128, 128)]],
            ctx_rows_v.at[pl.ds(t * 128, 128)], sem) for t in range(TPG)]
        for cp in cps:
            cp.wait()

        def elem_body(e, carry2):
            r0 = e * CTX
            acc = [ctx_rows_v[r0, pl.ds(16 * d, 16)] for d in range(ND)]
            for j in range(1, CTX):
                for d in range(ND):
                    acc[d] = acc[d] + ctx_rows_v[r0 + j, pl.ds(16 * d, 16)]
            for d in range(ND):
                cv_v[pl.ds(_mo8(e * DIM + 16 * d), 16)] = acc[d] * (1.0 / CTX)
            return carry2

        lax.fori_loop(0, CB, elem_body, 0)
        pltpu.sync_copy(cv_v, ctxv_out.at[pl.ds(_mo8((wid * BPW + c * CB) * DIM), CB * DIM)])
        return carry

    lax.fori_loop(0, NCHUNK, chunk_body, 0)


_k1 = functools.partial(
    pl.kernel,
    out_type=jax.ShapeDtypeStruct((BATCH * DIM,), jnp.float32),
    mesh=_SC_MESH,
    compiler_params=_SC_PARAMS,
    scratch_types=[
        pltpu.VMEM((BPW * CTX,), jnp.int32),
        pltpu.VMEM((ROWS, DIM), jnp.float32),
        pltpu.VMEM((CB * DIM,), jnp.float32),
        pltpu.SemaphoreType.DMA,
    ],
)(_k1_body)


def _k2_body(pos_idx_hbm, neg_idx_hbm, ctxv_hbm, out_emb,
             pos_out, neg_out,
             pos_idx_v, neg_idx_v, cv_v, neg_rows_v, pos_rows_v,
             pprod_v, nprod_v, pos_sc_v, neg_sc_v, sem, sem2):
    wid = lax.axis_index("s") * NC + lax.axis_index("c")
    pltpu.sync_copy(neg_idx_hbm.at[pl.ds(_mo8(wid * BPW * NEG), BPW * NEG)], neg_idx_v)
    pltpu.sync_copy(pos_idx_hbm.at[pl.ds(_mo8(wid * BPW), BPW)], pos_idx_v)
    iota16 = lax.iota(jnp.int32, 16)

    def issue_neg(c, half):
        off = _mo8(c * ROWS)
        for t in range(TPG):
            pltpu.async_copy(
                out_emb.at[neg_idx_v.at[pl.ds(off + t * 128, 128)]],
                neg_rows_v.at[pl.ds(_mo8(half * ROWS) + t * 128, 128)], sem)

    issue_neg(0, 0)

    def chunk_body(c, carry):
        half = lax.rem(c, 2)
        rbase = _mo8(half * ROWS)
        cps = [pltpu.async_copy(
            out_emb.at[pos_idx_v.at[pl.ds(_mo8(c * CB), CB)]], pos_rows_v, sem2),
            pltpu.async_copy(
            ctxv_hbm.at[pl.ds(_mo8((wid * BPW + c * CB) * DIM), CB * DIM)], cv_v, sem2)]

        @pl.when(c + 1 < NCHUNK)
        def _():
            issue_neg(c + 1, 1 - half)

        for t in range(TPG):
            pltpu.make_async_copy(
                out_emb.at[neg_idx_v.at[pl.ds(_mo8(c * ROWS) + t * 128, 128)]],
                neg_rows_v.at[pl.ds(rbase + t * 128, 128)], sem).wait()
        for cp in cps:
            cp.wait()

        def elem_body(e, carry2):
            cv = [cv_v[pl.ds(_mo8(e * DIM + 16 * d), 16)] for d in range(ND)]
            pr = [pos_rows_v[e, pl.ds(16 * d, 16)] for d in range(ND)]
            pp = cv[0] * pr[0] + cv[1] * pr[1] + cv[2] * pr[2] + cv[3] * pr[3]
            pprod_v[pl.ds(_mo8(e * 16), 16)] = pp
            r0 = rbase + e * NEG
            for k in range(NEG):
                nr = [neg_rows_v[r0 + k, pl.ds(16 * d, 16)] for d in range(ND)]
                np_ = cv[0] * nr[0] + cv[1] * nr[1] + cv[2] * nr[2] + cv[3] * nr[3]
                nprod_v[pl.ds(_mo8((e * NEG + k) * 16), 16)] = np_
            return carry2

        lax.fori_loop(0, CB, elem_body, 0, unroll=2)

        def pgroup(g, carry2):
            base = g * 256
            s = plsc.load_gather(pprod_v, [base + iota16 * 16])
            for d in range(1, 16):
                s = s + plsc.load_gather(pprod_v, [base + iota16 * 16 + d])
            pos_sc_v[pl.ds(_mo8(g * 16), 16)] = s
            return carry2

        lax.fori_loop(0, CB // 16, pgroup, 0)

        def ngroup(g, carry2):
            base = g * 256
            s = plsc.load_gather(nprod_v, [base + iota16 * 16])
            for d in range(1, 16):
                s = s + plsc.load_gather(nprod_v, [base + iota16 * 16 + d])
            neg_sc_v[pl.ds(_mo8(g * 16), 16)] = s
            return carry2

        lax.fori_loop(0, CB * NEG // 16, ngroup, 0, unroll=2)

        pltpu.sync_copy(pos_sc_v, pos_out.at[pl.ds(_mo8(wid * BPW + c * CB), CB)])
        pltpu.sync_copy(neg_sc_v,
                        neg_out.at[pl.ds(_mo8((wid * BPW + c * CB) * NEG), CB * NEG)])
        return carry

    lax.fori_loop(0, NCHUNK, chunk_body, 0)


_k2 = functools.partial(
    pl.kernel,
    out_type=(jax.ShapeDtypeStruct((BATCH,), jnp.float32),
              jax.ShapeDtypeStruct((BATCH * NEG,), jnp.float32)),
    mesh=_SC_MESH,
    compiler_params=_SC_PARAMS,
    scratch_types=[
        pltpu.VMEM((BPW,), jnp.int32),
        pltpu.VMEM((BPW * NEG,), jnp.int32),
        pltpu.VMEM((CB * DIM,), jnp.float32),
        pltpu.VMEM((2 * ROWS, DIM), jnp.float32),
        pltpu.VMEM((CB, DIM), jnp.float32),
        pltpu.VMEM((CB * 16,), jnp.float32),
        pltpu.VMEM((CB * NEG * 16,), jnp.float32),
        pltpu.VMEM((CB,), jnp.float32),
        pltpu.VMEM((CB * NEG,), jnp.float32),
        pltpu.SemaphoreType.DMA,
        pltpu.SemaphoreType.DMA,
    ],
)(_k2_body)


def _loss_body(pos_ref, neg_ref, out_ref):
    p = pos_ref[...]
    n = neg_ref[...]
    pls = jnp.log(1.0 / (1.0 + jnp.exp(-p)) + 1e-10)
    nls = jnp.log(1.0 / (1.0 + jnp.exp(n)) + 1e-10)
    total = -(jnp.sum(pls) + jnp.sum(nls)) / BATCH
    out_ref[...] = jnp.full((1, 1), total, jnp.float32)


def kernel(context_idxs, pos_target, neg_samples, in_embed, out_embed):
    ctx_flat = context_idxs.reshape(-1)
    neg_flat = neg_samples.reshape(-1)
    in_rm = lax.optimization_barrier(in_embed.reshape(-1)).reshape(VOCAB, DIM)
    out_rm = lax.optimization_barrier(out_embed.reshape(-1)).reshape(VOCAB, DIM)
    ctxv = _k1(ctx_flat, in_rm)
    pos_sc, neg_sc = _k2(pos_target, neg_flat, ctxv, out_rm)
    loss = pl.pallas_call(
        _loss_body,
        out_shape=jax.ShapeDtypeStruct((1, 1), jnp.float32),
    )(pos_sc.reshape(BATCH // 128, 128), neg_sc.reshape(BATCH * NEG // 128, 128))
    return loss[0, 0]


# speedup vs baseline: 1.0002x; 1.0002x over previous
import functools

import jax
import jax.numpy as jnp
from jax import lax
from jax.experimental import pallas as pl
from jax.experimental.pallas import tpu as pltpu
from jax.experimental.pallas import tpu_sc as plsc

VOCAB = 1000000
DIM = 64
BATCH = 16384
CTX = 20
NEG = 20

NC = 2
NS = 16
NW = NC * NS
BPW = BATCH // NW
CB = 32
NCHUNK = BPW // CB
ROWS = CB * CTX
TPG = ROWS // 128
ND = DIM // 16

_SC_MESH = plsc.VectorSubcoreMesh(core_axis_name="c", subcore_axis_name="s",
                                  num_cores=NC, num_subcores=NS)
_SC_PARAMS = pltpu.CompilerParams(needs_layout_passes=False,
                                  use_tc_tiling_on_sc=False)


def _mo8(x):
    return pl.multiple_of(x, 8)


def _k1_body(ctx_idx_hbm, in_emb, ctxv_out,
             ctx_idx_v, ctx_rows_v, cv_v, sem):
    wid = lax.axis_index("s") * NC + lax.axis_index("c")
    pltpu.sync_copy(ctx_idx_hbm.at[pl.ds(_mo8(wid * BPW * CTX), BPW * CTX)], ctx_idx_v)

    def chunk_body(c, carry):
        off = _mo8(c * ROWS)
        cps = [pltpu.async_copy(
            in_emb.at[ctx_idx_v.at[pl.ds(off + t * 128, 128)]],
            ctx_rows_v.at[pl.ds(t * 128, 128)], sem) for t in range(TPG)]
        for cp in cps:
            cp.wait()

        def elem_body(e, carry2):
            r0 = e * CTX
            acc = [ctx_rows_v[r0, pl.ds(16 * d, 16)] for d in range(ND)]
            for j in range(1, CTX):
                for d in range(ND):
                    acc[d] = acc[d] + ctx_rows_v[r0 + j, pl.ds(16 * d, 16)]
            for d in range(ND):
                cv_v[pl.ds(_mo8(e * DIM + 16 * d), 16)] = acc[d] * (1.0 / CTX)
            return carry2

        lax.fori_loop(0, CB, elem_body, 0)
        pltpu.sync_copy(cv_v, ctxv_out.at[pl.ds(_mo8((wid * BPW + c * CB) * DIM), CB * DIM)])
        return carry

    lax.fori_loop(0, NCHUNK, chunk_body, 0)


_k1 = functools.partial(
    pl.kernel,
    out_type=jax.ShapeDtypeStruct((BATCH * DIM,), jnp.float32),
    mesh=_SC_MESH,
    compiler_params=_SC_PARAMS,
    scratch_types=[
        pltpu.VMEM((BPW * CTX,), jnp.int32),
        pltpu.VMEM((ROWS, DIM), jnp.float32),
        pltpu.VMEM((CB * DIM,), jnp.float32),
        pltpu.SemaphoreType.DMA,
    ],
)(_k1_body)


def _k2_body(pos_idx_hbm, neg_idx_hbm, ctxv_hbm, out_emb,
             pos_out, neg_out,
             pos_idx_v, neg_idx_v, cv_v, neg_rows_v, pos_rows_v,
             pprod_v, nprod_v, pos_sc_v, neg_sc_v, sem, sem2):
    wid = lax.axis_index("s") * NC + lax.axis_index("c")
    pltpu.sync_copy(neg_idx_hbm.at[pl.ds(_mo8(wid * BPW * NEG), BPW * NEG)], neg_idx_v)
    pltpu.sync_copy(pos_idx_hbm.at[pl.ds(_mo8(wid * BPW), BPW)], pos_idx_v)
    iota16 = lax.iota(jnp.int32, 16)

    def issue_neg(c, half):
        off = _mo8(c * ROWS)
        for t in range(TPG):
            pltpu.async_copy(
                out_emb.at[neg_idx_v.at[pl.ds(off + t * 128, 128)]],
                neg_rows_v.at[pl.ds(_mo8(half * ROWS) + t * 128, 128)], sem)

    issue_neg(0, 0)

    def chunk_body(c, carry):
        half = lax.rem(c, 2)
        rbase = _mo8(half * ROWS)
        cps = [pltpu.async_copy(
            out_emb.at[pos_idx_v.at[pl.ds(_mo8(c * CB), CB)]], pos_rows_v, sem2),
            pltpu.async_copy(
            ctxv_hbm.at[pl.ds(_mo8((wid * BPW + c * CB) * DIM), CB * DIM)], cv_v, sem2)]

        @pl.when(c + 1 < NCHUNK)
        def _():
            issue_neg(c + 1, 1 - half)

        for t in range(TPG):
            pltpu.make_async_copy(
                out_emb.at[neg_idx_v.at[pl.ds(_mo8(c * ROWS) + t * 128, 128)]],
                neg_rows_v.at[pl.ds(rbase + t * 128, 128)], sem).wait()
        for cp in cps:
            cp.wait()

        def elem_body(e, carry2):
            cv = [cv_v[pl.ds(_mo8(e * DIM + 16 * d), 16)] for d in range(ND)]
            pr = [pos_rows_v[e, pl.ds(16 * d, 16)] for d in range(ND)]
            pp = cv[0] * pr[0] + cv[1] * pr[1] + cv[2] * pr[2] + cv[3] * pr[3]
            pprod_v[pl.ds(_mo8(e * 16), 16)] = pp
            r0 = rbase + e * NEG
            for k in range(NEG):
                nr = [neg_rows_v[r0 + k, pl.ds(16 * d, 16)] for d in range(ND)]
                np_ = cv[0] * nr[0] + cv[1] * nr[1] + cv[2] * nr[2] + cv[3] * nr[3]
                nprod_v[pl.ds(_mo8((e * NEG + k) * 16), 16)] = np_
            return carry2

        lax.fori_loop(0, CB, elem_body, 0, unroll=2)

        def pgroup(g, carry2):
            base = g * 256
            s = plsc.load_gather(pprod_v, [base + iota16 * 16])
            for d in range(1, 16):
                s = s + plsc.load_gather(pprod_v, [base + iota16 * 16 + d])
            pos_sc_v[pl.ds(_mo8(g * 16), 16)] = s
            return carry2

        lax.fori_loop(0, CB // 16, pgroup, 0)

        def ngroup(g, carry2):
            base = g * 256
            s = plsc.load_gather(nprod_v, [base + iota16 * 16])
            for d in range(1, 16):
                s = s + plsc.load_gather(nprod_v, [base + iota16 * 16 + d])
            neg_sc_v[pl.ds(_mo8(g * 16), 16)] = s
            return carry2

        lax.fori_loop(0, CB * NEG // 16, ngroup, 0, unroll=2)

        pltpu.sync_copy(pos_sc_v, pos_out.at[pl.ds(_mo8(wid * BPW + c * CB), CB)])
        pltpu.sync_copy(neg_sc_v,
                        neg_out.at[pl.ds(_mo8((wid * BPW + c * CB) * NEG), CB * NEG)])
        return carry

    lax.fori_loop(0, NCHUNK, chunk_body, 0)


_k2 = functools.partial(
    pl.kernel,
    out_type=(jax.ShapeDtypeStruct((BATCH,), jnp.float32),
              jax.ShapeDtypeStruct((BATCH * NEG,), jnp.float32)),
    mesh=_SC_MESH,
    compiler_params=_SC_PARAMS,
    scratch_types=[
        pltpu.VMEM((BPW,), jnp.int32),
        pltpu.VMEM((BPW * NEG,), jnp.int32),
        pltpu.VMEM((CB * DIM,), jnp.float32),
        pltpu.VMEM((2 * ROWS, DIM), jnp.float32),
        pltpu.VMEM((CB, DIM), jnp.float32),
        pltpu.VMEM((CB * 16,), jnp.float32),
        pltpu.VMEM((CB * NEG * 16,), jnp.float32),
        pltpu.VMEM((CB,), jnp.float32),
        pltpu.VMEM((CB * NEG,), jnp.float32),
        pltpu.SemaphoreType.DMA,
        pltpu.SemaphoreType.DMA,
    ],
)(_k2_body)


def _loss_body(pos_ref, neg_ref, out_ref):
    p = pos_ref[...]
    n = neg_ref[...]
    pls = jnp.log(1.0 / (1.0 + jnp.exp(-p)) + 1e-10)
    nls = jnp.log(1.0 / (1.0 + jnp.exp(n)) + 1e-10)
    total = -(jnp.sum(pls) + jnp.sum(nls)) / BATCH
    out_ref[...] = jnp.full((1, 1), total, jnp.float32)


def kernel(context_idxs, pos_target, neg_samples, in_embed, out_embed):
    ctx_flat = context_idxs.reshape(-1)
    neg_flat = neg_samples.reshape(-1)
    ctxv = _k1(ctx_flat, in_embed)
    pos_sc, neg_sc = _k2(pos_target, neg_flat, ctxv, out_embed)
    loss = pl.pallas_call(
        _loss_body,
        out_shape=jax.ShapeDtypeStruct((1, 1), jnp.float32),
    )(pos_sc.reshape(BATCH // 128, 128), neg_sc.reshape(BATCH * NEG // 128, 128))
    return loss[0, 0]


# speedup vs baseline: 1.0133x; 1.0130x over previous
import functools

import jax
import jax.numpy as jnp
from jax import lax
from jax.experimental import pallas as pl
from jax.experimental.pallas import tpu as pltpu
from jax.experimental.pallas import tpu_sc as plsc

VOCAB = 1000000
DIM = 64
BATCH = 16384
CTX = 20
NEG = 20

NC = 2
NS = 16
NW = NC * NS
BPW = BATCH // NW
CB = 32
NCHUNK = BPW // CB
ROWS = CB * CTX
TPG = ROWS // 128
ND = DIM // 16

_SC_MESH = plsc.VectorSubcoreMesh(core_axis_name="c", subcore_axis_name="s",
                                  num_cores=NC, num_subcores=NS)
_SC_PARAMS = pltpu.CompilerParams(needs_layout_passes=False,
                                  use_tc_tiling_on_sc=False)


def _mo8(x):
    return pl.multiple_of(x, 8)


def _k1_body(ctx_idx_hbm, in_emb, ctxv_out,
             ctx_idx_v, ctx_rows_v, cv_v, sem):
    wid = lax.axis_index("s") * NC + lax.axis_index("c")
    pltpu.sync_copy(ctx_idx_hbm.at[pl.ds(_mo8(wid * BPW * CTX), BPW * CTX)], ctx_idx_v)

    def chunk_body(c, carry):
        off = _mo8(c * ROWS)
        cps = [pltpu.async_copy(
            in_emb.at[ctx_idx_v.at[pl.ds(off + t * 128, 128)]],
            ctx_rows_v.at[pl.ds(t * 128, 128)], sem) for t in range(TPG)]
        for cp in cps:
            cp.wait()

        def elem_body(e, carry2):
            r0 = e * CTX
            acc = [ctx_rows_v[r0, pl.ds(16 * d, 16)] for d in range(ND)]
            for j in range(1, CTX):
                for d in range(ND):
                    acc[d] = acc[d] + ctx_rows_v[r0 + j, pl.ds(16 * d, 16)]
            for d in range(ND):
                cv_v[pl.ds(_mo8(e * DIM + 16 * d), 16)] = acc[d] * (1.0 / CTX)
            return carry2

        lax.fori_loop(0, CB, elem_body, 0)
        pltpu.sync_copy(cv_v, ctxv_out.at[pl.ds(_mo8((wid * BPW + c * CB) * DIM), CB * DIM)])
        return carry

    lax.fori_loop(0, NCHUNK, chunk_body, 0)


_k1 = functools.partial(
    pl.kernel,
    out_type=jax.ShapeDtypeStruct((BATCH * DIM,), jnp.float32),
    mesh=_SC_MESH,
    compiler_params=_SC_PARAMS,
    scratch_types=[
        pltpu.VMEM((BPW * CTX,), jnp.int32),
        pltpu.VMEM((ROWS, DIM), jnp.float32),
        pltpu.VMEM((CB * DIM,), jnp.float32),
        pltpu.SemaphoreType.DMA,
    ],
)(_k1_body)


def _k2_body(pos_idx_hbm, neg_idx_hbm, ctxv_hbm, out_emb,
             pos_out, neg_out,
             pos_idx_v, neg_idx_v, cv_v, neg_rows_v, pos_rows_v,
             pprod_v, nprod_v, pos_sc_v, neg_sc_v, sem, sem2):
    wid = lax.axis_index("s") * NC + lax.axis_index("c")
    pltpu.sync_copy(neg_idx_hbm.at[pl.ds(_mo8(wid * BPW * NEG), BPW * NEG)], neg_idx_v)
    pltpu.sync_copy(pos_idx_hbm.at[pl.ds(_mo8(wid * BPW), BPW)], pos_idx_v)
    iota16 = lax.iota(jnp.int32, 16)

    def issue_neg(c, half):
        off = _mo8(c * ROWS)
        for t in range(TPG):
            pltpu.async_copy(
                out_emb.at[neg_idx_v.at[pl.ds(off + t * 128, 128)]],
                neg_rows_v.at[pl.ds(_mo8(half * ROWS) + t * 128, 128)], sem)

    def issue_small(c, half):
        pltpu.async_copy(
            out_emb.at[pos_idx_v.at[pl.ds(_mo8(c * CB), CB)]],
            pos_rows_v.at[pl.ds(_mo8(half * CB), CB)], sem2)
        pltpu.async_copy(
            ctxv_hbm.at[pl.ds(_mo8((wid * BPW + c * CB) * DIM), CB * DIM)],
            cv_v.at[pl.ds(_mo8(half * CB * DIM), CB * DIM)], sem2)

    issue_neg(0, 0)
    issue_small(0, 0)

    def chunk_body(c, carry):
        half = lax.rem(c, 2)
        rbase = _mo8(half * ROWS)
        pbase = _mo8(half * CB)
        cvbase = _mo8(half * CB * DIM)

        @pl.when(c + 1 < NCHUNK)
        def _():
            issue_neg(c + 1, 1 - half)
            issue_small(c + 1, 1 - half)

        for t in range(TPG):
            pltpu.make_async_copy(
                out_emb.at[neg_idx_v.at[pl.ds(_mo8(c * ROWS) + t * 128, 128)]],
                neg_rows_v.at[pl.ds(rbase + t * 128, 128)], sem).wait()
        pltpu.make_async_copy(
            out_emb.at[pos_idx_v.at[pl.ds(_mo8(c * CB), CB)]],
            pos_rows_v.at[pl.ds(pbase, CB)], sem2).wait()
        pltpu.make_async_copy(
            ctxv_hbm.at[pl.ds(_mo8((wid * BPW + c * CB) * DIM), CB * DIM)],
            cv_v.at[pl.ds(cvbase, CB * DIM)], sem2).wait()

        def elem_body(e, carry2):
            cv = [cv_v[pl.ds(cvbase + _mo8(e * DIM + 16 * d), 16)] for d in range(ND)]
            pr = [pos_rows_v[pbase + e, pl.ds(16 * d, 16)] for d in range(ND)]
            pp = cv[0] * pr[0] + cv[1] * pr[1] + cv[2] * pr[2] + cv[3] * pr[3]
            pprod_v[pl.ds(_mo8(e * 16), 16)] = pp
            r0 = rbase + e * NEG
            for k in range(NEG):
                nr = [neg_rows_v[r0 + k, pl.ds(16 * d, 16)] for d in range(ND)]
                np_ = cv[0] * nr[0] + cv[1] * nr[1] + cv[2] * nr[2] + cv[3] * nr[3]
                nprod_v[pl.ds(_mo8((e * NEG + k) * 16), 16)] = np_
            return carry2

        lax.fori_loop(0, CB, elem_body, 0, unroll=2)

        def pgroup(g, carry2):
            base = g * 256
            s = plsc.load_gather(pprod_v, [base + iota16 * 16])
            for d in range(1, 16):
                s = s + plsc.load_gather(pprod_v, [base + iota16 * 16 + d])
            pos_sc_v[pl.ds(_mo8(g * 16), 16)] = s
            return carry2

        lax.fori_loop(0, CB // 16, pgroup, 0)

        def ngroup(g, carry2):
            base = g * 256
            s = plsc.load_gather(nprod_v, [base + iota16 * 16])
            for d in range(1, 16):
                s = s + plsc.load_gather(nprod_v, [base + iota16 * 16 + d])
            neg_sc_v[pl.ds(_mo8(g * 16), 16)] = s
            return carry2

        lax.fori_loop(0, CB * NEG // 16, ngroup, 0, unroll=2)

        pltpu.sync_copy(pos_sc_v, pos_out.at[pl.ds(_mo8(wid * BPW + c * CB), CB)])
        pltpu.sync_copy(neg_sc_v,
                        neg_out.at[pl.ds(_mo8((wid * BPW + c * CB) * NEG), CB * NEG)])
        return carry

    lax.fori_loop(0, NCHUNK, chunk_body, 0)


_k2 = functools.partial(
    pl.kernel,
    out_type=(jax.ShapeDtypeStruct((BATCH,), jnp.float32),
              jax.ShapeDtypeStruct((BATCH * NEG,), jnp.float32)),
    mesh=_SC_MESH,
    compiler_params=_SC_PARAMS,
    scratch_types=[
        pltpu.VMEM((BPW,), jnp.int32),
        pltpu.VMEM((BPW * NEG,), jnp.int32),
        pltpu.VMEM((2 * CB * DIM,), jnp.float32),
        pltpu.VMEM((2 * ROWS, DIM), jnp.float32),
        pltpu.VMEM((2 * CB, DIM), jnp.float32),
        pltpu.VMEM((CB * 16,), jnp.float32),
        pltpu.VMEM((CB * NEG * 16,), jnp.float32),
        pltpu.VMEM((CB,), jnp.float32),
        pltpu.VMEM((CB * NEG,), jnp.float32),
        pltpu.SemaphoreType.DMA,
        pltpu.SemaphoreType.DMA,
    ],
)(_k2_body)


def _loss_body(pos_ref, neg_ref, out_ref):
    p = pos_ref[...]
    n = neg_ref[...]
    pls = jnp.log(1.0 / (1.0 + jnp.exp(-p)) + 1e-10)
    nls = jnp.log(1.0 / (1.0 + jnp.exp(n)) + 1e-10)
    total = -(jnp.sum(pls) + jnp.sum(nls)) / BATCH
    out_ref[...] = jnp.full((1, 1), total, jnp.float32)


def kernel(context_idxs, pos_target, neg_samples, in_embed, out_embed):
    ctx_flat = context_idxs.reshape(-1)
    neg_flat = neg_samples.reshape(-1)
    ctxv = _k1(ctx_flat, in_embed)
    pos_sc, neg_sc = _k2(pos_target, neg_flat, ctxv, out_embed)
    loss = pl.pallas_call(
        _loss_body,
        out_shape=jax.ShapeDtypeStruct((1, 1), jnp.float32),
    )(pos_sc.reshape(BATCH // 128, 128), neg_sc.reshape(BATCH * NEG // 128, 128))
    return loss[0, 0]


# speedup vs baseline: 1.0137x; 1.0004x over previous
import functools

import jax
import jax.numpy as jnp
from jax import lax
from jax.experimental import pallas as pl
from jax.experimental.pallas import tpu as pltpu
from jax.experimental.pallas import tpu_sc as plsc

VOCAB = 1000000
DIM = 64
BATCH = 16384
CTX = 20
NEG = 20

NC = 2
NS = 16
NW = NC * NS
BPW = BATCH // NW
CB = 32
NCHUNK = BPW // CB
ROWS = CB * CTX
TPG = ROWS // 128
ND = DIM // 16

_SC_MESH = plsc.VectorSubcoreMesh(core_axis_name="c", subcore_axis_name="s",
                                  num_cores=NC, num_subcores=NS)
_SC_PARAMS = pltpu.CompilerParams(needs_layout_passes=False,
                                  use_tc_tiling_on_sc=False)


def _mo8(x):
    return pl.multiple_of(x, 8)


def _k1_body(ctx_idx_hbm, in_emb, ctxv_out,
             ctx_idx_v, ctx_rows_v, cv_v, sem):
    wid = lax.axis_index("s") * NC + lax.axis_index("c")
    pltpu.sync_copy(ctx_idx_hbm.at[pl.ds(_mo8(wid * BPW * CTX), BPW * CTX)], ctx_idx_v)

    def chunk_body(c, carry):
        off = _mo8(c * ROWS)
        cps = [pltpu.async_copy(
            in_emb.at[ctx_idx_v.at[pl.ds(off + t * 128, 128)]],
            ctx_rows_v.at[pl.ds(t * 128, 128)], sem) for t in range(TPG)]
        for cp in cps:
            cp.wait()

        def elem_body(e, carry2):
            r0 = e * CTX
            acc = [ctx_rows_v[r0, pl.ds(16 * d, 16)] for d in range(ND)]
            for j in range(1, CTX):
                for d in range(ND):
                    acc[d] = acc[d] + ctx_rows_v[r0 + j, pl.ds(16 * d, 16)]
            for d in range(ND):
                cv_v[pl.ds(_mo8(e * DIM + 16 * d), 16)] = acc[d] * (1.0 / CTX)
            return carry2

        lax.fori_loop(0, CB, elem_body, 0)
        pltpu.sync_copy(cv_v, ctxv_out.at[pl.ds(_mo8((wid * BPW + c * CB) * DIM), CB * DIM)])
        return carry

    lax.fori_loop(0, NCHUNK, chunk_body, 0)


_k1 = functools.partial(
    pl.kernel,
    out_type=jax.ShapeDtypeStruct((BATCH * DIM,), jnp.float32),
    mesh=_SC_MESH,
    compiler_params=_SC_PARAMS,
    scratch_types=[
        pltpu.VMEM((BPW * CTX,), jnp.int32),
        pltpu.VMEM((ROWS, DIM), jnp.float32),
        pltpu.VMEM((CB * DIM,), jnp.float32),
        pltpu.SemaphoreType.DMA,
    ],
)(_k1_body)


def _k2_body(pos_idx_hbm, neg_idx_hbm, ctxv_hbm, out_emb,
             pos_out, neg_out,
             pos_idx_v, neg_idx_v, cv_v, neg_rows_v, pos_rows_v,
             pprod_v, nprod_v, pos_sc_v, neg_sc_v, sem, sem2, sem3):
    wid = lax.axis_index("s") * NC + lax.axis_index("c")
    pltpu.sync_copy(neg_idx_hbm.at[pl.ds(_mo8(wid * BPW * NEG), BPW * NEG)], neg_idx_v)
    pltpu.sync_copy(pos_idx_hbm.at[pl.ds(_mo8(wid * BPW), BPW)], pos_idx_v)
    iota16 = lax.iota(jnp.int32, 16)

    def issue_neg(c, half):
        off = _mo8(c * ROWS)
        for t in range(TPG):
            pltpu.async_copy(
                out_emb.at[neg_idx_v.at[pl.ds(off + t * 128, 128)]],
                neg_rows_v.at[pl.ds(_mo8(half * ROWS) + t * 128, 128)], sem)

    def issue_small(c, half):
        pltpu.async_copy(
            out_emb.at[pos_idx_v.at[pl.ds(_mo8(c * CB), CB)]],
            pos_rows_v.at[pl.ds(_mo8(half * CB), CB)], sem2)
        pltpu.async_copy(
            ctxv_hbm.at[pl.ds(_mo8((wid * BPW + c * CB) * DIM), CB * DIM)],
            cv_v.at[pl.ds(_mo8(half * CB * DIM), CB * DIM)], sem2)

    issue_neg(0, 0)
    issue_small(0, 0)

    def score_drains(c, half):
        pltpu.make_async_copy(
            pos_sc_v.at[pl.ds(_mo8(half * CB), CB)],
            pos_out.at[pl.ds(_mo8(wid * BPW + c * CB), CB)], sem3).wait()
        pltpu.make_async_copy(
            neg_sc_v.at[pl.ds(_mo8(half * CB * NEG), CB * NEG)],
            neg_out.at[pl.ds(_mo8((wid * BPW + c * CB) * NEG), CB * NEG)],
            sem3).wait()

    def chunk_body(c, carry):
        half = lax.rem(c, 2)
        rbase = _mo8(half * ROWS)
        pbase = _mo8(half * CB)
        cvbase = _mo8(half * CB * DIM)
        sbase_p = _mo8(half * CB)
        sbase_n = _mo8(half * CB * NEG)

        @pl.when(c + 1 < NCHUNK)
        def _():
            issue_neg(c + 1, 1 - half)
            issue_small(c + 1, 1 - half)

        @pl.when(c >= 2)
        def _():
            score_drains(c - 2, half)

        for t in range(TPG):
            pltpu.make_async_copy(
                out_emb.at[neg_idx_v.at[pl.ds(_mo8(c * ROWS) + t * 128, 128)]],
                neg_rows_v.at[pl.ds(rbase + t * 128, 128)], sem).wait()
        pltpu.make_async_copy(
            out_emb.at[pos_idx_v.at[pl.ds(_mo8(c * CB), CB)]],
            pos_rows_v.at[pl.ds(pbase, CB)], sem2).wait()
        pltpu.make_async_copy(
            ctxv_hbm.at[pl.ds(_mo8((wid * BPW + c * CB) * DIM), CB * DIM)],
            cv_v.at[pl.ds(cvbase, CB * DIM)], sem2).wait()

        def elem_body(e, carry2):
            cv = [cv_v[pl.ds(cvbase + _mo8(e * DIM + 16 * d), 16)] for d in range(ND)]
            pr = [pos_rows_v[pbase + e, pl.ds(16 * d, 16)] for d in range(ND)]
            pp = cv[0] * pr[0] + cv[1] * pr[1] + cv[2] * pr[2] + cv[3] * pr[3]
            pprod_v[pl.ds(_mo8(e * 16), 16)] = pp
            r0 = rbase + e * NEG
            for k in range(NEG):
                nr = [neg_rows_v[r0 + k, pl.ds(16 * d, 16)] for d in range(ND)]
                np_ = cv[0] * nr[0] + cv[1] * nr[1] + cv[2] * nr[2] + cv[3] * nr[3]
                nprod_v[pl.ds(_mo8((e * NEG + k) * 16), 16)] = np_
            return carry2

        lax.fori_loop(0, CB, elem_body, 0, unroll=2)

        def pgroup(g, carry2):
            base = g * 256
            s = plsc.load_gather(pprod_v, [base + iota16 * 16])
            for d in range(1, 16):
                s = s + plsc.load_gather(pprod_v, [base + iota16 * 16 + d])
            pos_sc_v[pl.ds(_mo8(sbase_p + g * 16), 16)] = s
            return carry2

        lax.fori_loop(0, CB // 16, pgroup, 0)

        def ngroup(g, carry2):
            base = g * 256
            s = plsc.load_gather(nprod_v, [base + iota16 * 16])
            for d in range(1, 16):
                s = s + plsc.load_gather(nprod_v, [base + iota16 * 16 + d])
            neg_sc_v[pl.ds(_mo8(sbase_n + g * 16), 16)] = s
            return carry2

        lax.fori_loop(0, CB * NEG // 16, ngroup, 0, unroll=2)

        pltpu.async_copy(
            pos_sc_v.at[pl.ds(sbase_p, CB)],
            pos_out.at[pl.ds(_mo8(wid * BPW + c * CB), CB)], sem3)
        pltpu.async_copy(
            neg_sc_v.at[pl.ds(sbase_n, CB * NEG)],
            neg_out.at[pl.ds(_mo8((wid * BPW + c * CB) * NEG), CB * NEG)], sem3)
        return carry

    lax.fori_loop(0, NCHUNK, chunk_body, 0)
    score_drains(NCHUNK - 2, (NCHUNK - 2) % 2)
    score_drains(NCHUNK - 1, (NCHUNK - 1) % 2)


_k2 = functools.partial(
    pl.kernel,
    out_type=(jax.ShapeDtypeStruct((BATCH,), jnp.float32),
              jax.ShapeDtypeStruct((BATCH * NEG,), jnp.float32)),
    mesh=_SC_MESH,
    compiler_params=_SC_PARAMS,
    scratch_types=[
        pltpu.VMEM((BPW,), jnp.int32),
        pltpu.VMEM((BPW * NEG,), jnp.int32),
        pltpu.VMEM((2 * CB * DIM,), jnp.float32),
        pltpu.VMEM((2 * ROWS, DIM), jnp.float32),
        pltpu.VMEM((2 * CB, DIM), jnp.float32),
        pltpu.VMEM((CB * 16,), jnp.float32),
        pltpu.VMEM((CB * NEG * 16,), jnp.float32),
        pltpu.VMEM((2 * CB,), jnp.float32),
        pltpu.VMEM((2 * CB * NEG,), jnp.float32),
        pltpu.SemaphoreType.DMA,
        pltpu.SemaphoreType.DMA,
        pltpu.SemaphoreType.DMA,
    ],
)(_k2_body)


def _loss_body(pos_ref, neg_ref, out_ref):
    p = pos_ref[...]
    n = neg_ref[...]
    pls = jnp.log(1.0 / (1.0 + jnp.exp(-p)) + 1e-10)
    nls = jnp.log(1.0 / (1.0 + jnp.exp(n)) + 1e-10)
    total = -(jnp.sum(pls) + jnp.sum(nls)) / BATCH
    out_ref[...] = jnp.full((1, 1), total, jnp.float32)


def kernel(context_idxs, pos_target, neg_samples, in_embed, out_embed):
    ctx_flat = context_idxs.reshape(-1)
    neg_flat = neg_samples.reshape(-1)
    ctxv = _k1(ctx_flat, in_embed)
    pos_sc, neg_sc = _k2(pos_target, neg_flat, ctxv, out_embed)
    loss = pl.pallas_call(
        _loss_body,
        out_shape=jax.ShapeDtypeStruct((1, 1), jnp.float32),
    )(pos_sc.reshape(BATCH // 128, 128), neg_sc.reshape(BATCH * NEG // 128, 128))
    return loss[0, 0]
